# Initial kernel scaffold; baseline (speedup 1.0000x reference)
import jax
import jax.numpy as jnp
from jax.experimental import pallas as pl


def kernel(x, edge_index, edge_feat, Wu1, bu1, Wu2, bu2, Wv1, bv1, Wv2, bv2, We1, be1, We2, be2, Wn1, bn1, Wn2, bn2):
    raise NotImplementedError("write your pallas kernel here")


# speedup vs baseline: 1.3580x; 1.3580x over previous
import functools

import jax
import jax.numpy as jnp
from jax import lax
from jax.experimental import pallas as pl
from jax.experimental.pallas import tpu as pltpu
from jax.experimental.pallas import tpu_sc as plsc

N = 10000
E = 320000
D = 128
H = 128
ED = 16

NS = 16
CH = 128
CPT = 160
EPAD = NS * CPT * CH
NPAD = 10240
RPT = NPAD // NS
BN = 80
HALF = N // BN
HE0 = 2 * NPAD
TROWS = HE0 + EPAD
BE = 2560


def _silu(v):
    return v * (1.0 / (1.0 + jnp.exp(-v)))


def _node_mlps(x, Wu1, bu1, Wu2, bu2, Wv1, bv1, Wv2, bv2):
    def body(x_ref, wu1, bu1_, wu2, bu2_, wv1, bv1_, wv2, bv2_, t_ref):
        i = pl.program_id(0)
        pick = i < HALF
        w1 = jnp.where(pick, wu1[...], wv1[...])
        b1 = jnp.where(pick, bu1_[...], bv1_[...])
        w2 = jnp.where(pick, wu2[...], wv2[...])
        b2 = jnp.where(pick, bu2_[...], bv2_[...])
        h = _silu(jnp.dot(x_ref[...], w1, preferred_element_type=jnp.float32) + b1)
        t_ref[...] = _silu(jnp.dot(h, w2, preferred_element_type=jnp.float32) + b2)

    wspec = pl.BlockSpec((D, H), lambda i: (0, 0))
    bspec = pl.BlockSpec((1, H), lambda i: (0, 0))
    return pl.pallas_call(
        body,
        grid=(2 * HALF,),
        in_specs=[pl.BlockSpec((BN, D), lambda i: (i % HALF, 0)),
                  wspec, bspec, wspec, bspec, wspec, bspec, wspec, bspec],
        out_specs=pl.BlockSpec((BN, H), lambda i: (jnp.where(i < HALF, i, i + 3), 0)),
        out_shape=jax.ShapeDtypeStruct((TROWS, H), jnp.float32),
    )(x, Wu1, bu1.reshape(1, H), Wu2, bu2.reshape(1, H),
      Wv1, bv1.reshape(1, H), Wv2, bv2.reshape(1, H))


def _edge_mlp(tbl, ef, We1, be1, We2, be2):
    def body(tbl_ref, e_ref, w1, b1, w2, b2, he_ref):
        del tbl_ref
        h = _silu(jnp.dot(e_ref[...], w1[...], preferred_element_type=jnp.float32) + b1[...])
        he_ref[...] = _silu(jnp.dot(h, w2[...], preferred_element_type=jnp.float32) + b2[...])

    return pl.pallas_call(
        body,
        grid=(E // BE,),
        in_specs=[pl.BlockSpec(memory_space=pl.ANY),
                  pl.BlockSpec((BE, ED), lambda i: (i, 0)),
                  pl.BlockSpec((ED, H), lambda i: (0, 0)),
                  pl.BlockSpec((1, H), lambda i: (0, 0)),
                  pl.BlockSpec((H, H), lambda i: (0, 0)),
                  pl.BlockSpec((1, H), lambda i: (0, 0))],
        out_specs=pl.BlockSpec((BE, H), lambda i: (HE0 // BE + i, 0)),
        out_shape=jax.ShapeDtypeStruct((TROWS, H), jnp.float32),
        input_output_aliases={0: 0},
    )(tbl, ef, We1, be1.reshape(1, H), We2, be2.reshape(1, H))


def _sc_segment_sum(table, packed):
    mesh = plsc.VectorSubcoreMesh(core_axis_name="c", subcore_axis_name="s",
                                  num_cores=1)

    @functools.partial(
        pl.kernel,
        mesh=mesh,
        out_type=jax.ShapeDtypeStruct((NPAD, D), jnp.float32),
        scratch_types=[
            pltpu.VMEM((CPT, CH), jnp.int32),
            pltpu.VMEM((CH,), jnp.int32),
            pltpu.VMEM((CH,), jnp.int32),
            pltpu.VMEM((CH,), jnp.int32),
            pltpu.VMEM((CH,), jnp.int32),
            pltpu.VMEM((CH, D), jnp.float32),
            pltpu.VMEM((64, D), jnp.float32),
            pltpu.VMEM_SHARED((NPAD, D), jnp.float32),
            pltpu.SemaphoreType.DMA,
        ],
    )
    def k(tab_hbm, pk_hbm, un_out,
          pk_v, gsrc_v, gdst_v, ghe_v, sdst_v, rows_v, zbuf, un_sh, sem):
        t = lax.axis_index("s")

        def zstore(i, carry):
            r = i // (D // 16)
            col = (i % (D // 16)) * 16
            zbuf[r, pl.ds(col, 16)] = jnp.zeros((16,), jnp.float32)
            return carry
        lax.fori_loop(0, 64 * (D // 16), zstore, 0)

        def zcopy(j, carry):
            pltpu.sync_copy(zbuf, un_sh.at[pl.ds(t * RPT + j * 64, 64)])
            return carry
        lax.fori_loop(0, RPT // 64, zcopy, 0)

        pltpu.sync_copy(pk_hbm.at[pl.ds(t * CPT, CPT)], pk_v)

        plsc.subcore_barrier()

        lanes = lax.iota(jnp.int32, 16)

        def body(j, carry):
            he_base = HE0 + (t * CPT + j) * CH
            for kk in range(CH // 16):
                p = pk_v[j, pl.ds(kk * 16, 16)]
                d = lax.bitwise_and(p, 16383)
                s = lax.shift_right_logical(p, 14)
                sl = pl.ds(kk * 16, 16)
                sdst_v[sl] = d
                gsrc_v[sl] = s
                gdst_v[sl] = d + NPAD
                ghe_v[sl] = lanes + (he_base + kk * 16)
            pltpu.async_copy(tab_hbm.at[gsrc_v], rows_v, sem).wait()
            pltpu.sync_copy(rows_v, un_sh.at[sdst_v], add=True)
            pltpu.async_copy(tab_hbm.at[gdst_v], rows_v, sem).wait()
            pltpu.sync_copy(rows_v, un_sh.at[sdst_v], add=True)
            pltpu.async_copy(tab_hbm.at[ghe_v], rows_v, sem).wait()
            pltpu.sync_copy(rows_v, un_sh.at[sdst_v], add=True)
            return carry
        lax.fori_loop(0, CPT, body, 0)

        plsc.subcore_barrier()

        pltpu.sync_copy(un_sh.at[pl.ds(t * RPT, RPT)],
                        un_out.at[pl.ds(t * RPT, RPT)])

    return k(table, packed)


def _node_out_mlp(x, un, Wn1a, Wn1b, bn1, Wn2, bn2):
    B = 1000

    def body(x_ref, u0, wa, wb, b1, w2, b2, o_ref):
        tmp = _silu(jnp.dot(x_ref[...], wa[...], preferred_element_type=jnp.float32)
                    + jnp.dot(u0[...], wb[...], preferred_element_type=jnp.float32)
                    + b1[...])
        o_ref[...] = jnp.dot(tmp, w2[...], preferred_element_type=jnp.float32) + b2[...]

    wspec = pl.BlockSpec((H, H), lambda i: (0, 0))
    bspec = pl.BlockSpec((1, H), lambda i: (0, 0))
    return pl.pallas_call(
        body,
        grid=(N // B,),
        in_specs=[pl.BlockSpec((B, D), lambda i: (i, 0)),
                  pl.BlockSpec((B, H), lambda i: (i, 0)),
                  pl.BlockSpec((D, H), lambda i: (0, 0)), wspec, bspec, wspec, bspec],
        out_specs=pl.BlockSpec((B, H), lambda i: (i, 0)),
        out_shape=jax.ShapeDtypeStruct((N, H), jnp.float32),
    )(x, un, Wn1a, Wn1b, bn1.reshape(1, H), Wn2, bn2.reshape(1, H))


def kernel(x, edge_index, edge_feat,
           Wu1, bu1, Wu2, bu2,
           Wv1, bv1, Wv2, bv2,
           We1, be1, We2, be2,
           Wn1, bn1, Wn2, bn2):
    tbl = _node_mlps(x, Wu1, bu1, Wu2, bu2, Wv1, bv1, Wv2, bv2)
    tbl = _edge_mlp(tbl, edge_feat, We1, be1, We2, be2)

    pad = EPAD - E
    src_p = jnp.concatenate([edge_index[0], jnp.zeros((pad,), jnp.int32)])
    dst_p = jnp.concatenate([edge_index[1], jnp.full((pad,), N, jnp.int32)])
    packed = ((src_p << 14) | dst_p).reshape(EPAD // CH, CH)

    un = _sc_segment_sum(tbl, packed)

    return _node_out_mlp(x, un, Wn1[:D], Wn1[D:], bn1, Wn2, bn2)
